# baseline (speedup 1.0000x reference)
import functools

import jax
import jax.numpy as jnp
from jax import lax
from jax.experimental import pallas as pl
from jax.experimental.pallas import tpu as pltpu
from jax.experimental.pallas import tpu_sc as plsc

B = 16384
D = 64
NC = 2
NS = 16
L = 16
NW = NC * NS
BPW = B // NW
NCH = 4
CB = BPW // NCH
NG = CB // L


def _make_kernel():
    mesh = plsc.VectorSubcoreMesh(core_axis_name="c", subcore_axis_name="s")

    @functools.partial(
        pl.kernel,
        out_type=jax.ShapeDtypeStruct((B,), jnp.float32),
        mesh=mesh,
        compiler_params=pltpu.CompilerParams(
            needs_layout_passes=False, use_tc_tiling_on_sc=False),
        scratch_types=[
            pltpu.VMEM((NCH, CB), jnp.int32),
            pltpu.VMEM((NCH, CB), jnp.int32),
            pltpu.VMEM((BPW, D), jnp.float32),
            pltpu.VMEM((BPW, D), jnp.float32),
            pltpu.VMEM((BPW,), jnp.float32),
            pltpu.VMEM((BPW,), jnp.float32),
            pltpu.VMEM((BPW,), jnp.float32),
            pltpu.VMEM((L * L,), jnp.float32),
            pltpu.SemaphoreType.DMA,
            pltpu.SemaphoreType.DMA,
            pltpu.SemaphoreType.DMA,
            pltpu.SemaphoreType.DMA,
        ],
    )
    def mf(uids, iids, utab, itab, ubias, ibias, out,
           uidx, iidx, urows, irows, ubv, ibv, outv, tbuf,
           sem0, sem1, sem2, sem3):
        sems = [sem0, sem1, sem2, sem3]
        wid = lax.axis_index("s") * NC + lax.axis_index("c")
        base = wid * BPW

        copies = []
        for c in range(NCH):
            off = base + c * CB
            pltpu.sync_copy(uids.at[pl.ds(off, CB)], uidx.at[c])
            pltpu.sync_copy(iids.at[pl.ds(off, CB)], iidx.at[c])
            copies.append([
                pltpu.async_copy(utab.at[uidx.at[c]],
                                 urows.at[pl.ds(c * CB, CB)], sems[c]),
                pltpu.async_copy(itab.at[iidx.at[c]],
                                 irows.at[pl.ds(c * CB, CB)], sems[c]),
                pltpu.async_copy(ubias.at[uidx.at[c]],
                                 ubv.at[pl.ds(c * CB, CB)], sems[c]),
                pltpu.async_copy(ibias.at[iidx.at[c]],
                                 ibv.at[pl.ds(c * CB, CB)], sems[c]),
            ])

        lanes16 = lax.iota(jnp.int32, 16) * L

        for c in range(NCH):
            for cp in copies[c]:
                cp.wait()

            def group(gl, _, c=c):
                boff = c * CB + gl * L
                for r in range(L):
                    row = boff + r
                    acc = (urows[row, pl.ds(0, L)] * irows[row, pl.ds(0, L)]
                           + urows[row, pl.ds(L, L)] * irows[row, pl.ds(L, L)])
                    acc = acc + (urows[row, pl.ds(2 * L, L)]
                                 * irows[row, pl.ds(2 * L, L)]
                                 + urows[row, pl.ds(3 * L, L)]
                                 * irows[row, pl.ds(3 * L, L)])
                    plsc.store_scatter(tbuf, [lanes16 + r], acc)
                res = ubv[pl.ds(boff, L)] + ibv[pl.ds(boff, L)]
                for l in range(L):
                    res = res + tbuf[pl.ds(l * L, L)]
                outv[pl.ds(boff, L)] = res
                return 0

            lax.fori_loop(0, NG, group, 0)

        pltpu.sync_copy(outv, out.at[pl.ds(base, BPW)])

    return mf


_mf = _make_kernel()


def kernel(user_ids, item_ids, user_table, item_table, user_bias, item_bias):
    ub = user_bias.reshape((-1,))
    ib = item_bias.reshape((-1,))
    return _mf(user_ids, item_ids, user_table, item_table, ub, ib)
